# baseline (speedup 1.0000x reference)
import jax
import jax.numpy as jnp
from jax import lax
from jax.experimental import pallas as pl
from jax.experimental.pallas import tpu as pltpu
from jax.experimental.pallas import tpu_sc as plsc

N = 10000
K = 32
B = 128
EPS = 1e-08

NC = 1
NS = 16
NW = NC * NS
CHUNK = 8
N_PER_W = 640
N_PAD = NW * N_PER_W
N_CHUNKS = N_PER_W // CHUNK
NG = B // 16


def _body(w_hbm, idx_hbm, nx_hbm, ny_hbm, nz_hbm, g_hbm, out_hbm,
          idxv0, idxv1, wbuf0, wbuf1, ownw0, ownw1,
          xs, ys, zs, outb, gv, sem0, sem1):
    cid = lax.axis_index("c")
    sid = lax.axis_index("s")
    wid = sid * NC + cid
    sets = ((idxv0, wbuf0, ownw0, sem0), (idxv1, wbuf1, ownw1, sem1))

    pltpu.sync_copy(g_hbm, gv)
    g = gv[pl.ds(0, 16)][0]
    pltpu.sync_copy(nx_hbm, xs)
    pltpu.sync_copy(ny_hbm, ys)
    pltpu.sync_copy(nz_hbm, zs)

    def _dma_descs(tn, si):
        idxv, wbuf, ownw, sem = sets[si]
        vbase = wid * N_PER_W + tn * CHUNK
        descs = [
            pltpu.make_async_copy(w_hbm.at[idxv.at[j]],
                                  wbuf.at[pl.ds(j * 128, 128)], sem)
            for j in range(2)
        ]
        descs.append(
            pltpu.make_async_copy(w_hbm.at[pl.ds(vbase, CHUNK)], ownw, sem))
        return descs

    def issue(tn, si):
        idxv, _, _, _ = sets[si]
        pltpu.sync_copy(idx_hbm.at[wid, tn], idxv)
        for d in _dma_descs(tn, si):
            d.start()

    def wait(tn, si):
        for d in _dma_descs(tn, si):
            d.wait()

    def compute(t, si):
        _, wbuf, ownw, _ = sets[si]
        vbase = wid * N_PER_W + t * CHUNK
        idxv = sets[si][0]

        def vert_body(v, vcarry):
            vsplat = jnp.full((16,), vbase + v, jnp.int32)
            nxv = plsc.load_gather(xs, [vsplat])
            nyv = plsc.load_gather(ys, [vsplat])
            nzv = plsc.load_gather(zs, [vsplat])
            j = v // 4
            off0 = (v - 4 * j) * 32
            svec = []
            for kb in range(K // 16):
                nidx = idxv[j, pl.ds(off0 + kb * 16, 16)]
                sx = plsc.load_gather(xs, [nidx])
                sy = plsc.load_gather(ys, [nidx])
                sz = plsc.load_gather(zs, [nidx])
                svec.append(jnp.maximum(nxv * sx + nyv * sy + nzv * sz, 0.0))
            ssum = jnp.sum(svec[0], axis=0) + jnp.sum(svec[1], axis=0)
            inv = 1.0 / (jnp.full((16,), ssum, jnp.float32) + EPS)
            rowbase = v * K
            acc = [jnp.zeros((16,), jnp.float32) for _ in range(NG)]
            for k in range(K):
                s = svec[k // 16][k % 16]
                r = rowbase + k
                for gi in range(NG):
                    acc[gi] = acc[gi] + s * wbuf[r, pl.ds(gi * 16, 16)]
            rv = []
            for gi in range(NG):
                wv = ownw[v, pl.ds(gi * 16, 16)]
                rv.append(wv + g * (acc[gi] * inv - wv))
            m = rv[0]
            for gi in range(1, NG):
                m = jnp.maximum(m, rv[gi])
            mx = jnp.max(m, axis=0)
            ev = [jnp.exp(x - mx) for x in rv]
            tot = ev[0]
            for gi in range(1, NG):
                tot = tot + ev[gi]
            norm = 1.0 / jnp.full((16,), jnp.sum(tot, axis=0), jnp.float32)
            for gi in range(NG):
                outb[v, pl.ds(gi * 16, 16)] = ev[gi] * norm
            return vcarry

        lax.fori_loop(0, CHUNK, vert_body, 0)
        pltpu.sync_copy(outb, out_hbm.at[pl.ds(vbase, CHUNK)])

    issue(0, 0)
    issue(1, 1)

    def pair_body(p, carry):
        t0 = 2 * p
        wait(t0, 0)
        compute(t0, 0)
        issue(t0 + 2, 0)
        t1 = t0 + 1
        wait(t1, 1)
        compute(t1, 1)
        issue(t1 + 2, 1)
        return carry

    lax.fori_loop(0, N_CHUNKS // 2 - 1, pair_body, 0)
    wait(N_CHUNKS - 2, 0)
    compute(N_CHUNKS - 2, 0)
    wait(N_CHUNKS - 1, 1)
    compute(N_CHUNKS - 1, 1)


@jax.jit
def kernel(predicted_weights, vertex_neighbors, vertex_normals, gamma):
    w_tab = jnp.pad(predicted_weights, ((0, N_PAD - N), (0, 0)))
    npad = jnp.pad(vertex_normals, ((0, N_PAD - N), (0, 0)))
    nx, ny, nz = npad[:, 0], npad[:, 1], npad[:, 2]
    idx = vertex_neighbors.astype(jnp.int32)
    idx = jnp.pad(idx, ((0, N_PAD - N), (0, 0)))
    idx_r = idx.reshape(NW, N_CHUNKS, 2, 128)
    garr = jnp.broadcast_to(jnp.asarray(gamma, jnp.float32).reshape(()), (16,))

    mesh = plsc.VectorSubcoreMesh(core_axis_name="c", subcore_axis_name="s",
                                  num_cores=NC, num_subcores=NS)
    run = pl.kernel(
        _body,
        out_type=jax.ShapeDtypeStruct((N_PAD, B), jnp.float32),
        mesh=mesh,
        compiler_params=pltpu.CompilerParams(needs_layout_passes=False),
        scratch_types=[
            pltpu.VMEM((2, 128), jnp.int32),
            pltpu.VMEM((2, 128), jnp.int32),
            pltpu.VMEM((CHUNK * K, B), jnp.float32),
            pltpu.VMEM((CHUNK * K, B), jnp.float32),
            pltpu.VMEM((CHUNK, B), jnp.float32),
            pltpu.VMEM((CHUNK, B), jnp.float32),
            pltpu.VMEM((N_PAD,), jnp.float32),
            pltpu.VMEM((N_PAD,), jnp.float32),
            pltpu.VMEM((N_PAD,), jnp.float32),
            pltpu.VMEM((CHUNK, B), jnp.float32),
            pltpu.VMEM((16,), jnp.float32),
            pltpu.SemaphoreType.DMA,
            pltpu.SemaphoreType.DMA,
        ],
    )
    out = run(w_tab, idx_r, nx, ny, nz, garr)
    return out[:N]


# speedup vs baseline: 1.2447x; 1.2447x over previous
import jax
import jax.numpy as jnp
from jax import lax
from jax.experimental import pallas as pl
from jax.experimental.pallas import tpu as pltpu
from jax.experimental.pallas import tpu_sc as plsc

N = 10000
K = 32
B = 128
EPS = 1e-08

NC = 2
NS = 16
NW = NC * NS
CHUNK = 8
N_PER_W = 320
N_PAD = NW * N_PER_W
N_CHUNKS = N_PER_W // CHUNK
NG = B // 16


def _body(w_hbm, idx_hbm, nx_hbm, ny_hbm, nz_hbm, g_hbm, out_hbm,
          idxall, wbuf0, wbuf1, ownw0, ownw1,
          xs, ys, zs, outb, gv, sem0, sem1):
    cid = lax.axis_index("c")
    sid = lax.axis_index("s")
    wid = sid * NC + cid
    sets = ((wbuf0, ownw0, sem0), (wbuf1, ownw1, sem1))

    pltpu.sync_copy(g_hbm, gv)
    g = gv[pl.ds(0, 16)][0]
    pltpu.sync_copy(idx_hbm.at[wid], idxall)
    pltpu.sync_copy(nx_hbm, xs)
    pltpu.sync_copy(ny_hbm, ys)
    pltpu.sync_copy(nz_hbm, zs)

    def _dma_descs(tn, si):
        wbuf, ownw, sem = sets[si]
        vbase = wid * N_PER_W + tn * CHUNK
        descs = [
            pltpu.make_async_copy(w_hbm.at[idxall.at[tn * 4 + q]],
                                  wbuf.at[pl.ds(q * 64, 64)], sem)
            for q in range(4)
        ]
        descs.append(
            pltpu.make_async_copy(w_hbm.at[pl.ds(vbase, CHUNK)], ownw, sem))
        return descs

    def issue(tn, si):
        for d in _dma_descs(tn, si):
            d.start()

    def wait(tn, si):
        for d in _dma_descs(tn, si):
            d.wait()

    def compute(t, si):
        wbuf, ownw, _ = sets[si]
        vbase = wid * N_PER_W + t * CHUNK

        def vert_body(v, vcarry):
            vsplat = jnp.full((16,), vbase + v, jnp.int32)
            nxv = plsc.load_gather(xs, [vsplat])
            nyv = plsc.load_gather(ys, [vsplat])
            nzv = plsc.load_gather(zs, [vsplat])
            j = t * 4 + v // 2
            off0 = (v - 2 * (v // 2)) * 32
            svec = []
            for kb in range(K // 16):
                nidx = idxall[j, pl.ds(off0 + kb * 16, 16)]
                sx = plsc.load_gather(xs, [nidx])
                sy = plsc.load_gather(ys, [nidx])
                sz = plsc.load_gather(zs, [nidx])
                svec.append(jnp.maximum(nxv * sx + nyv * sy + nzv * sz, 0.0))
            ssum = jnp.sum(svec[0], axis=0) + jnp.sum(svec[1], axis=0)
            inv = 1.0 / (jnp.full((16,), ssum, jnp.float32) + EPS)
            rowbase = v * K
            acc = [jnp.zeros((16,), jnp.float32) for _ in range(NG)]
            for k in range(K):
                s = svec[k // 16][k % 16]
                r = rowbase + k
                for gi in range(NG):
                    acc[gi] = acc[gi] + s * wbuf[r, pl.ds(gi * 16, 16)]
            rv = []
            for gi in range(NG):
                wv = ownw[v, pl.ds(gi * 16, 16)]
                rv.append(wv + g * (acc[gi] * inv - wv))
            m = rv[0]
            for gi in range(1, NG):
                m = jnp.maximum(m, rv[gi])
            mx = jnp.max(m, axis=0)
            ev = [jnp.exp(x - mx) for x in rv]
            tot = ev[0]
            for gi in range(1, NG):
                tot = tot + ev[gi]
            norm = 1.0 / jnp.full((16,), jnp.sum(tot, axis=0), jnp.float32)
            for gi in range(NG):
                outb[v, pl.ds(gi * 16, 16)] = ev[gi] * norm
            return vcarry

        lax.fori_loop(0, CHUNK, vert_body, 0)
        pltpu.sync_copy(outb, out_hbm.at[pl.ds(vbase, CHUNK)])

    issue(0, 0)
    issue(1, 1)

    def pair_body(p, carry):
        t0 = 2 * p
        wait(t0, 0)
        compute(t0, 0)
        issue(t0 + 2, 0)
        t1 = t0 + 1
        wait(t1, 1)
        compute(t1, 1)
        issue(t1 + 2, 1)
        return carry

    lax.fori_loop(0, N_CHUNKS // 2 - 1, pair_body, 0)
    wait(N_CHUNKS - 2, 0)
    compute(N_CHUNKS - 2, 0)
    wait(N_CHUNKS - 1, 1)
    compute(N_CHUNKS - 1, 1)


@jax.jit
def kernel(predicted_weights, vertex_neighbors, vertex_normals, gamma):
    w_tab = jnp.pad(predicted_weights, ((0, N_PAD - N), (0, 0)))
    npad = jnp.pad(vertex_normals, ((0, N_PAD - N), (0, 0)))
    nx, ny, nz = npad[:, 0], npad[:, 1], npad[:, 2]
    idx = vertex_neighbors.astype(jnp.int32)
    idx = jnp.pad(idx, ((0, N_PAD - N), (0, 0)))
    idx_r = idx.reshape(NW, N_CHUNKS * 4, 64)
    garr = jnp.broadcast_to(jnp.asarray(gamma, jnp.float32).reshape(()), (16,))

    mesh = plsc.VectorSubcoreMesh(core_axis_name="c", subcore_axis_name="s",
                                  num_cores=NC, num_subcores=NS)
    run = pl.kernel(
        _body,
        out_type=jax.ShapeDtypeStruct((N_PAD, B), jnp.float32),
        mesh=mesh,
        compiler_params=pltpu.CompilerParams(needs_layout_passes=False),
        scratch_types=[
            pltpu.VMEM((N_CHUNKS * 4, 64), jnp.int32),
            pltpu.VMEM((CHUNK * K, B), jnp.float32),
            pltpu.VMEM((CHUNK * K, B), jnp.float32),
            pltpu.VMEM((CHUNK, B), jnp.float32),
            pltpu.VMEM((CHUNK, B), jnp.float32),
            pltpu.VMEM((N_PAD,), jnp.float32),
            pltpu.VMEM((N_PAD,), jnp.float32),
            pltpu.VMEM((N_PAD,), jnp.float32),
            pltpu.VMEM((CHUNK, B), jnp.float32),
            pltpu.VMEM((16,), jnp.float32),
            pltpu.SemaphoreType.DMA,
            pltpu.SemaphoreType.DMA,
        ],
    )
    out = run(w_tab, idx_r, nx, ny, nz, garr)
    return out[:N]


# speedup vs baseline: 1.3311x; 1.0694x over previous
import jax
import jax.numpy as jnp
from jax import lax
from jax.experimental import pallas as pl
from jax.experimental.pallas import tpu as pltpu
from jax.experimental.pallas import tpu_sc as plsc

N = 10000
K = 32
B = 128
EPS = 1e-08

NC = 2
NS = 16
CHUNK = 8
FAST_CID = 0
N_FAST_W = 496
N_SLOW_W = 144
N_PAD = NS * (N_FAST_W + N_SLOW_W)
MAXCH = N_FAST_W // CHUNK
NG = B // 16


def _body(w_hbm, idx_hbm, nx_hbm, ny_hbm, nz_hbm, g_hbm, out_hbm,
          idxv0, idxv1, wbuf0, wbuf1, ownw0, ownw1,
          xs, ys, zs, gv, sem0, sem1):
    cid = lax.axis_index("c")
    sid = lax.axis_index("s")
    sets = ((idxv0, wbuf0, ownw0, sem0), (idxv1, wbuf1, ownw1, sem1))

    pltpu.sync_copy(g_hbm, gv)
    g = gv[pl.ds(0, 16)][0]
    pltpu.sync_copy(nx_hbm, xs)
    pltpu.sync_copy(ny_hbm, ys)
    pltpu.sync_copy(nz_hbm, zs)

    @pl.when(cid == FAST_CID)
    def _fast_core():
        _worker(N_FAST_W, sid * N_FAST_W, sets, g,
                w_hbm, idx_hbm, out_hbm, xs, ys, zs)

    @pl.when(cid != FAST_CID)
    def _slow_core():
        _worker(N_SLOW_W, NS * N_FAST_W + sid * N_SLOW_W, sets, g,
                w_hbm, idx_hbm, out_hbm, xs, ys, zs)


def _worker(n_per_w, base, sets, g, w_hbm, idx_hbm, out_hbm, xs, ys, zs):
    nch = n_per_w // CHUNK
    base = pl.multiple_of(base, 16)

    def _dma_descs(tn, si):
        idxv, wbuf, ownw, sem = sets[si]
        vbase = pl.multiple_of(base + tn * CHUNK, 8)
        descs = [
            pltpu.make_async_copy(w_hbm.at[idxv.at[pl.ds(q * 128, 128)]],
                                  wbuf.at[pl.ds(q * 128, 128)], sem)
            for q in range(2)
        ]
        descs.append(
            pltpu.make_async_copy(w_hbm.at[pl.ds(vbase, CHUNK)], ownw, sem))
        return descs

    def issue(tn, si):
        idxv = sets[si][0]
        pltpu.sync_copy(
            idx_hbm.at[pl.ds(pl.multiple_of(base * K + tn * (CHUNK * K), 8),
                             CHUNK * K)], idxv)
        for d in _dma_descs(tn, si):
            d.start()

    def wait(tn, si):
        for d in _dma_descs(tn, si):
            d.wait()

    def compute(t, si):
        idxv, wbuf, ownw, _ = sets[si]
        vbase = pl.multiple_of(base + t * CHUNK, 8)

        def vert_body(v, vcarry):
            vsplat = jnp.full((16,), vbase + v, jnp.int32)
            nxv = plsc.load_gather(xs, [vsplat])
            nyv = plsc.load_gather(ys, [vsplat])
            nzv = plsc.load_gather(zs, [vsplat])
            svec = []
            for kb in range(K // 16):
                nidx = idxv[pl.ds(v * K + kb * 16, 16)]
                sx = plsc.load_gather(xs, [nidx])
                sy = plsc.load_gather(ys, [nidx])
                sz = plsc.load_gather(zs, [nidx])
                svec.append(jnp.maximum(nxv * sx + nyv * sy + nzv * sz, 0.0))
            ssum = jnp.sum(svec[0], axis=0) + jnp.sum(svec[1], axis=0)
            inv = 1.0 / (jnp.full((16,), ssum, jnp.float32) + EPS)
            rowbase = v * K
            acc = [jnp.zeros((16,), jnp.float32) for _ in range(NG)]
            for k in range(K):
                s = svec[k // 16][k % 16]
                r = rowbase + k
                for gi in range(NG):
                    acc[gi] = acc[gi] + s * wbuf[r, pl.ds(gi * 16, 16)]
            rv = []
            for gi in range(NG):
                wv = ownw[v, pl.ds(gi * 16, 16)]
                rv.append(wv + g * (acc[gi] * inv - wv))
            m = rv[0]
            for gi in range(1, NG):
                m = jnp.maximum(m, rv[gi])
            mx = jnp.max(m, axis=0)
            ev = [jnp.exp(x - mx) for x in rv]
            tot = ev[0]
            for gi in range(1, NG):
                tot = tot + ev[gi]
            norm = 1.0 / jnp.full((16,), jnp.sum(tot, axis=0), jnp.float32)
            for gi in range(NG):
                ownw[v, pl.ds(gi * 16, 16)] = ev[gi] * norm
            return vcarry

        lax.fori_loop(0, CHUNK, vert_body, 0)
        pltpu.sync_copy(ownw, out_hbm.at[pl.ds(vbase, CHUNK)])

    issue(0, 0)
    issue(1, 1)

    def pair_body(p, carry):
        t0 = 2 * p
        wait(t0, 0)
        compute(t0, 0)

        @pl.when(p < nch // 2 - 1)
        def _i0():
            issue(t0 + 2, 0)

        t1 = t0 + 1
        wait(t1, 1)
        compute(t1, 1)

        @pl.when(p < nch // 2 - 1)
        def _i1():
            issue(t1 + 2, 1)

        return carry

    lax.fori_loop(0, nch // 2, pair_body, 0)


@jax.jit
def kernel(predicted_weights, vertex_neighbors, vertex_normals, gamma):
    w_tab = jnp.pad(predicted_weights, ((0, N_PAD - N), (0, 0)))
    npad = jnp.pad(vertex_normals, ((0, N_PAD - N), (0, 0)))
    nx, ny, nz = npad[:, 0], npad[:, 1], npad[:, 2]
    idx = vertex_neighbors.astype(jnp.int32)
    idx = jnp.pad(idx, ((0, N_PAD - N), (0, 0)))
    idx_r = idx.reshape(-1)
    garr = jnp.broadcast_to(jnp.asarray(gamma, jnp.float32).reshape(()), (16,))

    mesh = plsc.VectorSubcoreMesh(core_axis_name="c", subcore_axis_name="s",
                                  num_cores=NC, num_subcores=NS)
    run = pl.kernel(
        _body,
        out_type=jax.ShapeDtypeStruct((N_PAD, B), jnp.float32),
        mesh=mesh,
        compiler_params=pltpu.CompilerParams(needs_layout_passes=False),
        scratch_types=[
            pltpu.VMEM((CHUNK * K,), jnp.int32),
            pltpu.VMEM((CHUNK * K,), jnp.int32),
            pltpu.VMEM((CHUNK * K, B), jnp.float32),
            pltpu.VMEM((CHUNK * K, B), jnp.float32),
            pltpu.VMEM((CHUNK, B), jnp.float32),
            pltpu.VMEM((CHUNK, B), jnp.float32),
            pltpu.VMEM((N_PAD,), jnp.float32),
            pltpu.VMEM((N_PAD,), jnp.float32),
            pltpu.VMEM((N_PAD,), jnp.float32),
            pltpu.VMEM((16,), jnp.float32),
            pltpu.SemaphoreType.DMA,
            pltpu.SemaphoreType.DMA,
        ],
    )
    out = run(w_tab, idx_r, nx, ny, nz, garr)
    return out[:N]
